# baseline (speedup 1.0000x reference)
import functools

import jax
import jax.numpy as jnp
from jax.experimental import pallas as pl
from jax.experimental.pallas import tpu as pltpu


def _tc_body(x_ref, perm_ref, signs_ref, o_ref, p_scr, *, rb):
    d = p_scr.shape[0]

    @pl.when(pl.program_id(0) == 0)
    def _():
        ii = jax.lax.broadcasted_iota(jnp.int32, (d, d), 0)
        eq = ii == perm_ref[...][None, :]
        p_scr[...] = jnp.where(eq, signs_ref[...][None, :], jnp.float32(0.0))

    p = p_scr[...]
    for r in range(rb):
        o_ref[r] = jax.lax.dot_general(
            x_ref[r], p, (((1,), (0,)), ((), ())),
            precision=jax.lax.Precision.HIGHEST,
            preferred_element_type=jnp.float32)


def kernel(x, perm, signs):
    b, s, d = x.shape
    rb = 16
    grid = (b // rb,)
    return pl.pallas_call(
        functools.partial(_tc_body, rb=rb),
        grid=grid,
        in_specs=[
            pl.BlockSpec((rb, s, d), lambda i: (i, 0, 0)),
            pl.BlockSpec((d,), lambda i: (0,)),
            pl.BlockSpec((d,), lambda i: (0,)),
        ],
        out_specs=pl.BlockSpec((rb, s, d), lambda i: (i, 0, 0)),
        out_shape=jax.ShapeDtypeStruct((b, s, d), jnp.float32),
        scratch_shapes=[pltpu.VMEM((d, d), jnp.float32)],
    )(x, perm, signs)
